# Initial kernel scaffold; baseline (speedup 1.0000x reference)
import jax
import jax.numpy as jnp
from jax.experimental import pallas as pl


def kernel(z0, z1, codebook0, codebook1):
    raise NotImplementedError("write your pallas kernel here")


# speedup vs baseline: 2.1576x; 2.1576x over previous
import functools

import jax
import jax.numpy as jnp
from jax.experimental import pallas as pl
from jax.experimental.pallas import tpu as pltpu


def _vq_body(z_ref, cb_ref, q_ref, perp_ref, counts_ref, *, n_tokens, K):
    b = pl.program_id(0)
    t = pl.program_id(1)
    nb = pl.num_programs(0)
    nt = pl.num_programs(1)

    z_blk = z_ref[0]
    cb = cb_ref[...]

    z2 = jnp.sum(z_blk * z_blk, axis=0)
    cb2 = jnp.sum(cb * cb, axis=1)
    scores = jax.lax.dot_general(
        cb, z_blk,
        dimension_numbers=(((1,), (0,)), ((), ())),
        preferred_element_type=jnp.float32,
    )
    d = (z2[None, :] - 2.0 * scores) + cb2[:, None]

    idx = jnp.argmin(d, axis=0)
    p = (jax.lax.broadcasted_iota(jnp.int32, d.shape, 0)
         == idx[None, :]).astype(jnp.float32)

    q_blk = jax.lax.dot_general(
        cb, p,
        dimension_numbers=(((0,), (0,)), ((), ())),
        preferred_element_type=jnp.float32,
        precision=jax.lax.Precision.HIGHEST,
    )
    q_ref[0] = q_blk

    @pl.when(jnp.logical_and(b == 0, t == 0))
    def _init():
        counts_ref[...] = jnp.zeros_like(counts_ref)

    counts_ref[0, :] += jnp.sum(p, axis=1)

    @pl.when(jnp.logical_and(b == nb - 1, t == nt - 1))
    def _finalize():
        pm = counts_ref[0, :] / float(n_tokens)
        perp = jnp.exp(-jnp.sum(pm * jnp.log(pm + 1e-10)))
        perp_ref[...] = perp.reshape(1, 1)


def _vq_pair(z, cb, tt):
    B, C, T = z.shape
    K = cb.shape[0]
    nt = T // tt
    body = functools.partial(_vq_body, n_tokens=B * T, K=K)
    q, perp = pl.pallas_call(
        body,
        grid=(B, nt),
        in_specs=[
            pl.BlockSpec((1, C, tt), lambda b, t: (b, 0, t)),
            pl.BlockSpec((K, C), lambda b, t: (0, 0)),
        ],
        out_specs=[
            pl.BlockSpec((1, C, tt), lambda b, t: (b, 0, t)),
            pl.BlockSpec((1, 1), lambda b, t: (0, 0)),
        ],
        out_shape=[
            jax.ShapeDtypeStruct((B, C, T), jnp.float32),
            jax.ShapeDtypeStruct((1, 1), jnp.float32),
        ],
        scratch_shapes=[pltpu.VMEM((1, K), jnp.float32)],
        compiler_params=pltpu.CompilerParams(
            dimension_semantics=("arbitrary", "arbitrary"),
        ),
    )(z, cb)
    return q, perp.reshape(())


def kernel(z0, z1, codebook0, codebook1):
    q0, perp0 = _vq_pair(z0, codebook0, tt=1024)
    q1, perp1 = _vq_pair(z1, codebook1, tt=1024)
    return (q0, q1, q0, q1, perp0, perp1)


# speedup vs baseline: 3.2516x; 1.5071x over previous
import functools

import jax
import jax.numpy as jnp
from jax.experimental import pallas as pl
from jax.experimental.pallas import tpu as pltpu


def _vq_body(z_ref, cb_ref, q_ref, perp_ref, counts_ref, *, n_tokens, K):
    b = pl.program_id(0)
    t = pl.program_id(1)
    nb = pl.num_programs(0)
    nt = pl.num_programs(1)

    z_blk = z_ref[0]
    cb = cb_ref[...]
    tt = z_blk.shape[1]

    z2 = jnp.sum(z_blk * z_blk, axis=0)
    cb2 = jnp.sum(cb * cb, axis=1)
    scores = jax.lax.dot_general(
        cb, z_blk,
        dimension_numbers=(((1,), (0,)), ((), ())),
        preferred_element_type=jnp.float32,
    )
    d = (z2[None, :] - 2.0 * scores) + cb2[:, None]

    idx = jnp.argmin(d, axis=0)
    p = (jax.lax.broadcasted_iota(jnp.int32, d.shape, 0)
         == idx[None, :]).astype(jnp.bfloat16)

    cb_hi = cb.astype(jnp.bfloat16)
    cb_lo = (cb - cb_hi.astype(jnp.float32)).astype(jnp.bfloat16)
    dn = (((0,), (0,)), ((), ()))
    q_hi = jax.lax.dot_general(cb_hi, p, dn, preferred_element_type=jnp.float32)
    q_lo = jax.lax.dot_general(cb_lo, p, dn, preferred_element_type=jnp.float32)
    q_ref[0] = q_hi + q_lo

    ones_col = jnp.ones((tt, 8), dtype=jnp.bfloat16)
    pc = jax.lax.dot_general(
        p, ones_col,
        dimension_numbers=(((1,), (0,)), ((), ())),
        preferred_element_type=jnp.float32,
    )

    @pl.when(jnp.logical_and(b == 0, t == 0))
    def _init():
        counts_ref[...] = jnp.zeros_like(counts_ref)

    counts_ref[...] += pc

    @pl.when(jnp.logical_and(b == nb - 1, t == nt - 1))
    def _finalize():
        pm = jnp.sum(counts_ref[...], axis=1) / float(8 * n_tokens)
        perp = jnp.exp(-jnp.sum(pm * jnp.log(pm + 1e-10)))
        perp_ref[...] = perp.reshape(1, 1)


def _vq_pair(z, cb, tt):
    B, C, T = z.shape
    K = cb.shape[0]
    nt = T // tt
    body = functools.partial(_vq_body, n_tokens=B * T, K=K)
    q, perp = pl.pallas_call(
        body,
        grid=(B, nt),
        in_specs=[
            pl.BlockSpec((1, C, tt), lambda b, t: (b, 0, t)),
            pl.BlockSpec((K, C), lambda b, t: (0, 0)),
        ],
        out_specs=[
            pl.BlockSpec((1, C, tt), lambda b, t: (b, 0, t)),
            pl.BlockSpec((1, 1), lambda b, t: (0, 0)),
        ],
        out_shape=[
            jax.ShapeDtypeStruct((B, C, T), jnp.float32),
            jax.ShapeDtypeStruct((1, 1), jnp.float32),
        ],
        scratch_shapes=[pltpu.VMEM((K, 8), jnp.float32)],
        compiler_params=pltpu.CompilerParams(
            dimension_semantics=("arbitrary", "arbitrary"),
        ),
    )(z, cb)
    return q, perp.reshape(())


def kernel(z0, z1, codebook0, codebook1):
    q0, perp0 = _vq_pair(z0, codebook0, tt=1024)
    q1, perp1 = _vq_pair(z1, codebook1, tt=1024)
    return (q0, q1, q0, q1, perp0, perp1)


# speedup vs baseline: 4.3443x; 1.3361x over previous
import functools

import jax
import jax.numpy as jnp
from jax.experimental import pallas as pl
from jax.experimental.pallas import tpu as pltpu


def _vq_body(z_ref, cb_ref, q_ref, perp_ref, counts_ref, *, n_tokens, K):
    b = pl.program_id(0)
    t = pl.program_id(1)
    nb = pl.num_programs(0)
    nt = pl.num_programs(1)

    z_blk = z_ref[0]
    cb = cb_ref[...]
    tt = z_blk.shape[1]

    z2 = jnp.sum(z_blk * z_blk, axis=0)
    cb2 = jnp.sum(cb * cb, axis=1)
    scores = jax.lax.dot_general(
        -2.0 * cb, z_blk,
        dimension_numbers=(((1,), (0,)), ((), ())),
        preferred_element_type=jnp.float32,
    )
    d = (z2[None, :] + scores) + cb2[:, None]

    idx = jnp.argmin(d, axis=0)
    p = (jax.lax.broadcasted_iota(jnp.int32, d.shape, 0)
         == idx[None, :]).astype(jnp.bfloat16)

    q_ref[0] = jax.lax.dot_general(
        cb.astype(jnp.bfloat16), p,
        dimension_numbers=(((0,), (0,)), ((), ())),
        preferred_element_type=jnp.float32,
    )

    ones_col = jnp.ones((tt, 8), dtype=jnp.bfloat16)
    pc = jax.lax.dot_general(
        p, ones_col,
        dimension_numbers=(((1,), (0,)), ((), ())),
        preferred_element_type=jnp.float32,
    )

    @pl.when(jnp.logical_and(b == 0, t == 0))
    def _init():
        counts_ref[...] = jnp.zeros_like(counts_ref)

    counts_ref[...] += pc

    @pl.when(jnp.logical_and(b == nb - 1, t == nt - 1))
    def _finalize():
        pm = jnp.sum(counts_ref[...], axis=1) / float(8 * n_tokens)
        perp = jnp.exp(-jnp.sum(pm * jnp.log(pm + 1e-10)))
        perp_ref[...] = perp.reshape(1, 1)


def _vq_pair(z, cb, tt):
    B, C, T = z.shape
    K = cb.shape[0]
    nt = T // tt
    body = functools.partial(_vq_body, n_tokens=B * T, K=K)
    q, perp = pl.pallas_call(
        body,
        grid=(B, nt),
        in_specs=[
            pl.BlockSpec((1, C, tt), lambda b, t: (b, 0, t)),
            pl.BlockSpec((K, C), lambda b, t: (0, 0)),
        ],
        out_specs=[
            pl.BlockSpec((1, C, tt), lambda b, t: (b, 0, t)),
            pl.BlockSpec((1, 1), lambda b, t: (0, 0)),
        ],
        out_shape=[
            jax.ShapeDtypeStruct((B, C, T), jnp.float32),
            jax.ShapeDtypeStruct((1, 1), jnp.float32),
        ],
        scratch_shapes=[pltpu.VMEM((K, 8), jnp.float32)],
        compiler_params=pltpu.CompilerParams(
            dimension_semantics=("arbitrary", "arbitrary"),
        ),
    )(z, cb)
    return q, perp.reshape(())


def kernel(z0, z1, codebook0, codebook1):
    q0, perp0 = _vq_pair(z0, codebook0, tt=2048)
    q1, perp1 = _vq_pair(z1, codebook1, tt=2048)
    return (q0, q1, q0, q1, perp0, perp1)


# speedup vs baseline: 4.5312x; 1.0430x over previous
import functools

import jax
import jax.numpy as jnp
from jax import lax
from jax.experimental import pallas as pl
from jax.experimental.pallas import tpu as pltpu
from jax.experimental.pallas import tpu_sc as plsc

_NC = 2
_NS = 16
_NW = _NC * _NS
_LANES = 16


def _vq_body(z_ref, cb_ref, q_ref, idx_ref):
    z_blk = z_ref[0]
    cb = cb_ref[...]

    z2 = jnp.sum(z_blk * z_blk, axis=0)
    cb2 = jnp.sum(cb * cb, axis=1)
    scores = jax.lax.dot_general(
        -2.0 * cb, z_blk,
        dimension_numbers=(((1,), (0,)), ((), ())),
        preferred_element_type=jnp.float32,
    )
    d = (z2[None, :] + scores) + cb2[:, None]

    idx = jnp.argmin(d, axis=0)
    idx_ref[0, 0] = idx
    p = (jax.lax.broadcasted_iota(jnp.int32, d.shape, 0)
         == idx[None, :]).astype(jnp.bfloat16)

    q_ref[0] = jax.lax.dot_general(
        cb.astype(jnp.bfloat16), p,
        dimension_numbers=(((0,), (0,)), ((), ())),
        preferred_element_type=jnp.float32,
    )


def _vq_pair(z, cb, tt):
    B, C, T = z.shape
    K = cb.shape[0]
    nt = T // tt
    q, idx = pl.pallas_call(
        _vq_body,
        grid=(B, nt),
        in_specs=[
            pl.BlockSpec((1, C, tt), lambda b, t: (b, 0, t)),
            pl.BlockSpec((K, C), lambda b, t: (0, 0)),
        ],
        out_specs=[
            pl.BlockSpec((1, C, tt), lambda b, t: (b, 0, t)),
            pl.BlockSpec((1, 1, tt), lambda b, t: (b, 0, t)),
        ],
        out_shape=[
            jax.ShapeDtypeStruct((B, C, T), jnp.float32),
            jax.ShapeDtypeStruct((B, 1, T), jnp.int32),
        ],
        compiler_params=pltpu.CompilerParams(
            dimension_semantics=("arbitrary", "arbitrary"),
        ),
    )(z, cb)
    return q, idx.reshape(B * T)


def _make_sc_hist(n0, n1, K):
    c0, c1 = n0 // _NW, n1 // _NW
    mesh = plsc.VectorSubcoreMesh(core_axis_name="c", subcore_axis_name="s")

    @functools.partial(
        pl.kernel, mesh=mesh,
        out_type=[
            jax.ShapeDtypeStruct((_NW, K), jnp.float32),
            jax.ShapeDtypeStruct((_NW, K), jnp.float32),
        ],
        scratch_types=[
            pltpu.VMEM((max(c0, c1),), jnp.int32),
            pltpu.VMEM((_LANES, K), jnp.float32),
            pltpu.VMEM((K,), jnp.float32),
        ],
        compiler_params=pltpu.CompilerParams(needs_layout_passes=False),
    )
    def hist_kernel(idx0_hbm, idx1_hbm, out0_hbm, out1_hbm,
                    idx_v, bins2d, binrow):
        wid = lax.axis_index("s") * _NC + lax.axis_index("c")
        lane_iota = lax.iota(jnp.int32, _LANES)
        ones16 = jnp.ones((_LANES,), jnp.float32)
        zeros16 = jnp.zeros((_LANES,), jnp.float32)
        ncol = K // _LANES

        def one_pair(idx_hbm, out_hbm, n):
            pltpu.sync_copy(idx_hbm.at[pl.ds(wid * n, n)],
                            idx_v.at[pl.ds(0, n)])

            def zero_row(r, _):
                def zero_col(c, _):
                    bins2d[r, pl.ds(c * _LANES, _LANES)] = zeros16
                    return 0
                return lax.fori_loop(0, ncol, zero_col, 0)
            lax.fori_loop(0, _LANES, zero_row, 0)

            def scat(i, _):
                v = idx_v[pl.ds(i * _LANES, _LANES)]
                plsc.addupdate_scatter(bins2d, [lane_iota, v], ones16)
                return 0
            lax.fori_loop(0, n // _LANES, scat, 0)

            def red_col(c, _):
                def red_row(r, acc):
                    return acc + bins2d[r, pl.ds(c * _LANES, _LANES)]
                acc = lax.fori_loop(0, _LANES, red_row, zeros16)
                binrow[pl.ds(c * _LANES, _LANES)] = acc
                return 0
            lax.fori_loop(0, ncol, red_col, 0)

            pltpu.sync_copy(binrow, out_hbm.at[wid])

        one_pair(idx0_hbm, out0_hbm, c0)
        one_pair(idx1_hbm, out1_hbm, c1)

    return hist_kernel


def _perp_body(c0_ref, c1_ref, perp_ref, *, n0, n1):
    pm0 = jnp.sum(c0_ref[...], axis=0) / float(n0)
    pm1 = jnp.sum(c1_ref[...], axis=0) / float(n1)
    p0 = jnp.exp(-jnp.sum(pm0 * jnp.log(pm0 + 1e-10)))
    p1 = jnp.exp(-jnp.sum(pm1 * jnp.log(pm1 + 1e-10)))
    perp_ref[...] = jnp.concatenate(
        [p0.reshape(1, 1), p1.reshape(1, 1)], axis=1)


def _perplexities(part0, part1, n0, n1):
    NW, K = part0.shape
    body = functools.partial(_perp_body, n0=n0, n1=n1)
    perp = pl.pallas_call(
        body,
        in_specs=[
            pl.BlockSpec((NW, K), lambda: (0, 0)),
            pl.BlockSpec((NW, K), lambda: (0, 0)),
        ],
        out_specs=pl.BlockSpec((1, 2), lambda: (0, 0)),
        out_shape=jax.ShapeDtypeStruct((1, 2), jnp.float32),
    )(part0, part1)
    return perp[0, 0], perp[0, 1]


def kernel(z0, z1, codebook0, codebook1):
    K = codebook0.shape[0]
    q0, idx0 = _vq_pair(z0, codebook0, tt=2048)
    q1, idx1 = _vq_pair(z1, codebook1, tt=2048)
    hist = _make_sc_hist(idx0.shape[0], idx1.shape[0], K)
    part0, part1 = hist(idx0, idx1)
    perp0, perp1 = _perplexities(part0, part1, idx0.shape[0], idx1.shape[0])
    return (q0, q1, q0, q1, perp0, perp1)


# speedup vs baseline: 5.5484x; 1.2245x over previous
import functools

import jax
import jax.numpy as jnp
from jax.experimental import pallas as pl
from jax.experimental.pallas import tpu as pltpu


def _vq_block(z_blk, cb, q_ref, cnt_ref):
    z2 = jnp.sum(z_blk * z_blk, axis=0)
    cb2 = jnp.sum(cb * cb, axis=1)
    scores = jax.lax.dot_general(
        -2.0 * cb, z_blk,
        dimension_numbers=(((1,), (0,)), ((), ())),
        preferred_element_type=jnp.float32,
    )
    d = (z2[None, :] + scores) + cb2[:, None]

    idx = jnp.argmin(d, axis=0)
    p = (jax.lax.broadcasted_iota(jnp.int32, d.shape, 0)
         == idx[None, :]).astype(jnp.bfloat16)

    q_ref[0] = jax.lax.dot_general(
        cb.astype(jnp.bfloat16), p,
        dimension_numbers=(((0,), (0,)), ((), ())),
        preferred_element_type=jnp.float32,
    )

    f = p
    while f.shape[1] > 128:
        h = f.shape[1] // 2
        f = f[:, :h] + f[:, h:]
    cnt_ref[...] += f.astype(jnp.float32)


def _vq_body(z0_ref, z1_ref, cb_ref, q0_ref, q1_ref, perp_ref,
             cnt0_ref, cnt1_ref, *, nt0, n0, n1):
    b = pl.program_id(0)
    t = pl.program_id(1)
    nb = pl.num_programs(0)
    nt = pl.num_programs(1)
    cb = cb_ref[0]

    @pl.when(jnp.logical_and(b == 0, t == 0))
    def _init():
        cnt0_ref[...] = jnp.zeros_like(cnt0_ref)
        cnt1_ref[...] = jnp.zeros_like(cnt1_ref)

    @pl.when(t < nt0)
    def _pair0():
        _vq_block(z0_ref[0], cb, q0_ref, cnt0_ref)

    @pl.when(t >= nt0)
    def _pair1():
        _vq_block(z1_ref[0], cb, q1_ref, cnt1_ref)

    @pl.when(jnp.logical_and(b == nb - 1, t == nt - 1))
    def _finalize():
        pm0 = jnp.sum(cnt0_ref[...], axis=1) / float(n0)
        pm1 = jnp.sum(cnt1_ref[...], axis=1) / float(n1)
        p0 = jnp.exp(-jnp.sum(pm0 * jnp.log(pm0 + 1e-10)))
        p1 = jnp.exp(-jnp.sum(pm1 * jnp.log(pm1 + 1e-10)))
        perp_ref[...] = jnp.concatenate(
            [p0.reshape(1, 1), p1.reshape(1, 1)], axis=1)


def kernel(z0, z1, codebook0, codebook1):
    B, C, T0 = z0.shape
    T1 = z1.shape[2]
    K = codebook0.shape[0]
    tt = 2048
    nt0, nt1 = T0 // tt, T1 // tt
    nt = nt0 + nt1
    cbs = jnp.stack([codebook0, codebook1])
    body = functools.partial(_vq_body, nt0=nt0, n0=B * T0, n1=B * T1)
    q0, q1, perp = pl.pallas_call(
        body,
        grid=(B, nt),
        in_specs=[
            pl.BlockSpec((1, C, tt),
                         lambda b, t: (b, 0, jnp.minimum(t, nt0 - 1))),
            pl.BlockSpec((1, C, tt),
                         lambda b, t: (b, 0, jnp.maximum(t - nt0, 0))),
            pl.BlockSpec((1, K, C),
                         lambda b, t: ((t >= nt0).astype(jnp.int32), 0, 0)),
        ],
        out_specs=[
            pl.BlockSpec((1, C, tt),
                         lambda b, t: (b, 0, jnp.minimum(t, nt0 - 1))),
            pl.BlockSpec((1, C, tt),
                         lambda b, t: (b, 0, jnp.maximum(t - nt0, 0))),
            pl.BlockSpec((1, 2), lambda b, t: (0, 0)),
        ],
        out_shape=[
            jax.ShapeDtypeStruct((B, C, T0), jnp.float32),
            jax.ShapeDtypeStruct((B, C, T1), jnp.float32),
            jax.ShapeDtypeStruct((1, 2), jnp.float32),
        ],
        scratch_shapes=[
            pltpu.VMEM((K, 128), jnp.float32),
            pltpu.VMEM((K, 128), jnp.float32),
        ],
        compiler_params=pltpu.CompilerParams(
            dimension_semantics=("arbitrary", "arbitrary"),
        ),
    )(z0, z1, cbs)
    return (q0, q1, q0, q1, perp[0, 0], perp[0, 1])
